# baseline (speedup 1.0000x reference)
import functools

import jax
import jax.numpy as jnp
from jax import lax
from jax.experimental import pallas as pl
from jax.experimental.pallas import tpu as pltpu
from jax.experimental.pallas import tpu_sc as plsc

IN_DIM = 2048
L1_DIM = 1024
L2_DIM = 512
D1_LO = 2048
D2_LO = 3072
N_EDGES = 262144
BATCH = 256
OUT_DIM = 64

NUM_CORES = 2
NUM_SUBCORES = 16
R1 = L1_DIM // NUM_CORES
R2 = L2_DIM // NUM_CORES
W1_WORDS = R1 * IN_DIM
W2_WORDS = R2 * D2_LO
DUMP = W1_WORDS + W2_WORDS
ACC = DUMP + 128
EPT = N_EDGES // NUM_SUBCORES
CH = 2048
NCHUNK = EPT // CH
GRP = 128
NGRP = CH // GRP
LANES = 16
ZBUF = 4096
NCHUNK_ALL = N_EDGES // CH


W1SPAN = W1_WORDS // NUM_SUBCORES
W2SPAN = W2_WORDS // NUM_SUBCORES


def _build_w_kernel(pk_hbm, wt_hbm, w1_hbm, w2_hbm,
                    acc_sh, buf_v, wt_v, idx_v, zero_v, sem):
    s = lax.axis_index("s")

    def zfill(i, _):
        zero_v[pl.ds(i * LANES, LANES)] = jnp.zeros((LANES,), jnp.float32)
        return 0
    lax.fori_loop(0, ZBUF // LANES, zfill, 0)

    def zero_phase():
        b1 = s * W1SPAN
        def z1(i, _):
            pltpu.sync_copy(zero_v, acc_sh.at[pl.ds(b1 + i * ZBUF, ZBUF)])
            return 0
        lax.fori_loop(0, W1SPAN // ZBUF, z1, 0)
        b2 = W1_WORDS + s * W2SPAN
        def z2(i, _):
            pltpu.sync_copy(zero_v, acc_sh.at[pl.ds(b2 + i * ZBUF, ZBUF)])
            return 0
        lax.fori_loop(0, W2SPAN // ZBUF, z2, 0)
        pltpu.sync_copy(zero_v.at[pl.ds(0, ACC - DUMP)],
                        acc_sh.at[pl.ds(DUMP, ACC - DUMP)])

    def scan_half(d1_base, d2_base):
        def chunk_body(ch, _):
            pltpu.sync_copy(pk_hbm.at[s * NCHUNK + ch], buf_v)
            pltpu.sync_copy(wt_hbm.at[pl.ds((s * NCHUNK + ch) * CH, CH)],
                            wt_v)

            def vec_body(v, _):
                sv = buf_v[0, pl.ds(v * LANES, LANES)]
                dv = buf_v[1, pl.ds(v * LANES, LANES)]
                r1 = dv - d1_base
                m1 = (r1 >= 0) & (r1 < R1) & (sv < IN_DIM)
                r2 = dv - d2_base
                m2 = (r2 >= 0) & (r2 < R2) & (sv < D2_LO)
                iv = jnp.where(m1, r1 * IN_DIM + sv,
                               jnp.where(m2, W1_WORDS + r2 * D2_LO + sv,
                                         DUMP + (sv & 127)))
                idx_v[v // (GRP // LANES),
                      pl.ds((v % (GRP // LANES)) * LANES, LANES)] = iv
                return 0
            lax.fori_loop(0, CH // LANES, vec_body, 0)

            def fire(g, _):
                pltpu.async_copy(wt_v.at[pl.ds(g * GRP, GRP)],
                                 acc_sh.at[idx_v.at[g]], sem, add=True)
                return 0
            lax.fori_loop(0, NGRP, fire, 0)
            def drain(g, _):
                pltpu.make_async_copy(wt_v.at[pl.ds(g * GRP, GRP)],
                                      acc_sh.at[idx_v.at[g]], sem).wait()
                return 0
            lax.fori_loop(0, NGRP, drain, 0)
            return 0
        lax.fori_loop(0, NCHUNK, chunk_body, 0)

    def copyout_half(half):
        pltpu.sync_copy(acc_sh.at[pl.ds(s * W1SPAN, W1SPAN)],
                        w1_hbm.at[pl.ds(half * W1_WORDS + s * W1SPAN,
                                        W1SPAN)])
        pltpu.sync_copy(acc_sh.at[pl.ds(W1_WORDS + s * W2SPAN, W2SPAN)],
                        w2_hbm.at[pl.ds(half * W2_WORDS + s * W2SPAN,
                                        W2SPAN)])

    zero_phase()
    plsc.subcore_barrier()
    for half in (0, 1):
        scan_half(D1_LO + half * R1, D2_LO + half * R2)
        plsc.subcore_barrier()
        copyout_half(half)
        if half == 0:
            zero_phase()
            plsc.subcore_barrier()


_build_w = functools.partial(
    pl.kernel,
    mesh=plsc.VectorSubcoreMesh(core_axis_name="c", subcore_axis_name="s",
                                num_cores=1),
    out_type=[
        jax.ShapeDtypeStruct((L1_DIM * IN_DIM,), jnp.float32),
        jax.ShapeDtypeStruct((L2_DIM * D2_LO,), jnp.float32),
    ],
    scratch_types=[
        pltpu.VMEM_SHARED((ACC,), jnp.float32),
        pltpu.VMEM((2, CH), jnp.int32),
        pltpu.VMEM((CH,), jnp.float32),
        pltpu.VMEM((NGRP, GRP), jnp.int32),
        pltpu.VMEM((ZBUF,), jnp.float32),
        pltpu.SemaphoreType.DMA,
    ],
)(_build_w_kernel)


def _ln(z, g, b):
    mu = jnp.mean(z, axis=-1, keepdims=True)
    var = jnp.mean((z - mu) ** 2, axis=-1, keepdims=True)
    return (z - mu) * lax.rsqrt(var + 1e-5) * g + b


def _dense_body(x_ref, w1t_ref, w2t_ref, b1_ref, b2_ref, g1_ref, bb1_ref,
                g2_ref, bb2_ref, hw1_ref, hb1_ref, hw2_ref, hb2_ref,
                stacked_ref, avg_ref):
    x = x_ref[...]
    dn = (((1,), (1,)), ((), ()))
    z1 = lax.dot_general(x, w1t_ref[...], dn,
                         preferred_element_type=jnp.float32) + b1_ref[...]
    act1 = jnp.tanh(_ln(z1, g1_ref[...], bb1_ref[...]))
    p1 = lax.dot_general(act1, hw1_ref[...], dn,
                         preferred_element_type=jnp.float32) + hb1_ref[...]
    z2 = (lax.dot_general(x, w2t_ref[:, :IN_DIM], dn,
                          preferred_element_type=jnp.float32)
          + lax.dot_general(act1, w2t_ref[:, IN_DIM:], dn,
                            preferred_element_type=jnp.float32)
          + b2_ref[...])
    act2 = jnp.tanh(_ln(z2, g2_ref[...], bb2_ref[...]))
    p2 = lax.dot_general(act2, hw2_ref[...], dn,
                         preferred_element_type=jnp.float32) + hb2_ref[...]
    stacked_ref[0] = p1
    stacked_ref[1] = p2
    avg_ref[...] = (p1 + p2) * 0.5


def kernel(x, edge_index, weight, bias, g1, b1, g2, b2, hw1, hb1, hw2, hb2):
    packed = edge_index.reshape(2, NCHUNK_ALL, CH).transpose(1, 0, 2)
    w1t, w2t = _build_w(packed, weight)
    w1t = w1t.reshape(L1_DIM, IN_DIM)
    w2t = w2t.reshape(L2_DIM, D2_LO)
    b1r = bias[0:L1_DIM].reshape(1, L1_DIM)
    b2r = bias[L1_DIM:L1_DIM + L2_DIM].reshape(1, L2_DIM)
    stacked, avg = pl.pallas_call(
        _dense_body,
        out_shape=(
            jax.ShapeDtypeStruct((2, BATCH, OUT_DIM), jnp.float32),
            jax.ShapeDtypeStruct((BATCH, OUT_DIM), jnp.float32),
        ),
    )(x, w1t, w2t, b1r, b2r,
      g1.reshape(1, L1_DIM), b1.reshape(1, L1_DIM),
      g2.reshape(1, L2_DIM), b2.reshape(1, L2_DIM),
      hw1, hb1.reshape(1, OUT_DIM), hw2, hb2.reshape(1, OUT_DIM))
    return (avg, stacked)


# speedup vs baseline: 1.4841x; 1.4841x over previous
import functools

import jax
import jax.numpy as jnp
from jax import lax
from jax.experimental import pallas as pl
from jax.experimental.pallas import tpu as pltpu
from jax.experimental.pallas import tpu_sc as plsc

IN_DIM = 2048
L1_DIM = 1024
L2_DIM = 512
D1_LO = 2048
D2_LO = 3072
N_EDGES = 262144
BATCH = 256
OUT_DIM = 64

NUM_CORES = 2
NUM_SUBCORES = 16
R1 = L1_DIM // NUM_CORES
R2 = L2_DIM // NUM_CORES
W1_WORDS = R1 * IN_DIM
W2_WORDS = R2 * D2_LO
DUMP = W1_WORDS + W2_WORDS
ACC = DUMP + 128
EPT = N_EDGES // NUM_SUBCORES
CH = 2048
NCHUNK = EPT // CH
GRP = 128
NGRP = CH // GRP
LANES = 16
ZBUF = 4096
NCHUNK_ALL = N_EDGES // CH


def _build_w_kernel(pk_hbm, wt_hbm, w1_hbm, w2_hbm,
                    acc_sh, buf_v, wt_v, idx_v, zero_v, sem):
    c = lax.axis_index("c")
    s = lax.axis_index("s")

    def zfill(i, _):
        zero_v[pl.ds(i * LANES, LANES)] = jnp.zeros((LANES,), jnp.float32)
        return 0
    lax.fori_loop(0, ZBUF // LANES, zfill, 0)
    span = ACC // NUM_SUBCORES
    base = s * span
    def zcopy(i, _):
        pltpu.sync_copy(zero_v, acc_sh.at[pl.ds(base + i * ZBUF, ZBUF)])
        return 0
    lax.fori_loop(0, span // ZBUF, zcopy, 0)
    pltpu.sync_copy(zero_v, acc_sh.at[pl.ds(base + span - ZBUF, ZBUF)])

    plsc.subcore_barrier()

    d1_base = D1_LO + c * R1
    d2_base = D2_LO + c * R2

    def chunk_body(ch, _):
        pltpu.async_copy(pk_hbm.at[s * NCHUNK + ch], buf_v, sem)
        pltpu.async_copy(wt_hbm.at[pl.ds((s * NCHUNK + ch) * CH, CH)],
                         wt_v, sem)
        pltpu.make_async_copy(pk_hbm.at[s * NCHUNK + ch], buf_v, sem).wait()
        pltpu.make_async_copy(wt_hbm.at[pl.ds((s * NCHUNK + ch) * CH, CH)],
                              wt_v, sem).wait()

        def vec_body(v, _):
            sv = buf_v[0, pl.ds(v * LANES, LANES)]
            dv = buf_v[1, pl.ds(v * LANES, LANES)]
            r1 = dv - d1_base
            m1 = (r1 >= 0) & (r1 < R1) & (sv < IN_DIM)
            r2 = dv - d2_base
            m2 = (r2 >= 0) & (r2 < R2) & (sv < D2_LO)
            iv = jnp.where(m1, r1 * IN_DIM + sv,
                           jnp.where(m2, W1_WORDS + r2 * D2_LO + sv,
                                     DUMP + (sv & 127)))
            idx_v[v // (GRP // LANES),
                  pl.ds((v % (GRP // LANES)) * LANES, LANES)] = iv
            return 0
        lax.fori_loop(0, CH // LANES, vec_body, 0)

        def fire(g, _):
            pltpu.async_copy(wt_v.at[pl.ds(g * GRP, GRP)],
                             acc_sh.at[idx_v.at[g]], sem, add=True)
            return 0
        lax.fori_loop(0, NGRP, fire, 0)
        def drain(g, _):
            pltpu.make_async_copy(wt_v.at[pl.ds(g * GRP, GRP)],
                                  acc_sh.at[idx_v.at[g]], sem).wait()
            return 0
        lax.fori_loop(0, NGRP, drain, 0)
        return 0
    lax.fori_loop(0, NCHUNK, chunk_body, 0)

    plsc.subcore_barrier()

    w1_span = W1_WORDS // NUM_SUBCORES
    pltpu.sync_copy(acc_sh.at[pl.ds(s * w1_span, w1_span)],
                    w1_hbm.at[pl.ds(c * W1_WORDS + s * w1_span, w1_span)])
    w2_span = W2_WORDS // NUM_SUBCORES
    pltpu.sync_copy(acc_sh.at[pl.ds(W1_WORDS + s * w2_span, w2_span)],
                    w2_hbm.at[pl.ds(c * W2_WORDS + s * w2_span, w2_span)])


_build_w = functools.partial(
    pl.kernel,
    mesh=plsc.VectorSubcoreMesh(core_axis_name="c", subcore_axis_name="s"),
    out_type=[
        jax.ShapeDtypeStruct((L1_DIM * IN_DIM,), jnp.float32),
        jax.ShapeDtypeStruct((L2_DIM * D2_LO,), jnp.float32),
    ],
    scratch_types=[
        pltpu.VMEM_SHARED((ACC,), jnp.float32),
        pltpu.VMEM((2, CH), jnp.int32),
        pltpu.VMEM((CH,), jnp.float32),
        pltpu.VMEM((NGRP, GRP), jnp.int32),
        pltpu.VMEM((ZBUF,), jnp.float32),
        pltpu.SemaphoreType.DMA,
    ],
)(_build_w_kernel)


def _ln(z, g, b):
    mu = jnp.mean(z, axis=-1, keepdims=True)
    var = jnp.mean((z - mu) ** 2, axis=-1, keepdims=True)
    return (z - mu) * lax.rsqrt(var + 1e-5) * g + b


def _dense_body(x_ref, w1t_ref, w2t_ref, b1_ref, b2_ref, g1_ref, bb1_ref,
                g2_ref, bb2_ref, hw1_ref, hb1_ref, hw2_ref, hb2_ref,
                stacked_ref, avg_ref):
    x = x_ref[...]
    dn = (((1,), (1,)), ((), ()))
    z1 = lax.dot_general(x, w1t_ref[...], dn,
                         preferred_element_type=jnp.float32) + b1_ref[...]
    act1 = jnp.tanh(_ln(z1, g1_ref[...], bb1_ref[...]))
    p1 = lax.dot_general(act1, hw1_ref[...], dn,
                         preferred_element_type=jnp.float32) + hb1_ref[...]
    z2 = (lax.dot_general(x, w2t_ref[:, :IN_DIM], dn,
                          preferred_element_type=jnp.float32)
          + lax.dot_general(act1, w2t_ref[:, IN_DIM:], dn,
                            preferred_element_type=jnp.float32)
          + b2_ref[...])
    act2 = jnp.tanh(_ln(z2, g2_ref[...], bb2_ref[...]))
    p2 = lax.dot_general(act2, hw2_ref[...], dn,
                         preferred_element_type=jnp.float32) + hb2_ref[...]
    stacked_ref[0] = p1
    stacked_ref[1] = p2
    avg_ref[...] = (p1 + p2) * 0.5


def kernel(x, edge_index, weight, bias, g1, b1, g2, b2, hw1, hb1, hw2, hb2):
    packed = edge_index.reshape(2, NCHUNK_ALL, CH).transpose(1, 0, 2)
    w1t, w2t = _build_w(packed, weight)
    w1t = w1t.reshape(L1_DIM, IN_DIM)
    w2t = w2t.reshape(L2_DIM, D2_LO)
    b1r = bias[0:L1_DIM].reshape(1, L1_DIM)
    b2r = bias[L1_DIM:L1_DIM + L2_DIM].reshape(1, L2_DIM)
    stacked, avg = pl.pallas_call(
        _dense_body,
        out_shape=(
            jax.ShapeDtypeStruct((2, BATCH, OUT_DIM), jnp.float32),
            jax.ShapeDtypeStruct((BATCH, OUT_DIM), jnp.float32),
        ),
    )(x, w1t, w2t, b1r, b2r,
      g1.reshape(1, L1_DIM), b1.reshape(1, L1_DIM),
      g2.reshape(1, L2_DIM), b2.reshape(1, L2_DIM),
      hw1, hb1.reshape(1, OUT_DIM), hw2, hb2.reshape(1, OUT_DIM))
    return (avg, stacked)
